# baseline (speedup 1.0000x reference)
import jax
import jax.numpy as jnp
from jax.experimental import pallas as pl
from jax.experimental.pallas import tpu as pltpu

_B = 4096
_K = 1024
_BM = 1024
_BN = 1024
_JT = _B // _BM
_IT = _B // _BN
_N = _JT * _IT


def _body(y_ref, z_ref, t1_ref, t10_ref,
          da_ref, db_ref, d_ref, ny_ref, nx_ref, acc_ref):
    t = pl.program_id(0)
    ones_row = jnp.ones((1, _K), dtype=jnp.float32)
    ones_col = jnp.ones((_BN, 1), dtype=jnp.float32)

    jm = jnp.minimum(t, _N - 1) // _IT
    sm = jnp.minimum(t, _N - 1) - jm * _IT
    im = jax.lax.rem(jm + sm, _IT)

    te = jnp.maximum(t - 1, 0)
    je = te // _IT
    se = te - je * _IT
    ie = jax.lax.rem(je + se, _IT)

    @pl.when(sm == 0)
    def _():
        ny2 = jax.lax.dot_general(
            y_ref[...] * y_ref[...], ones_row, (((1,), (1,)), ((), ())),
            preferred_element_type=jnp.float32)
        ny_ref[jax.lax.rem(jm, 2)] = jnp.sqrt(ny2)

    @pl.when(jm == 0)
    def _():
        nx2 = jax.lax.dot_general(
            ones_row, z_ref[...] * z_ref[...], (((1,), (1,)), ((), ())),
            preferred_element_type=jnp.float32)
        nx_ref[im] = jnp.sqrt(nx2)

    def _mm(dst_ref):
        dst_ref[...] = jax.lax.dot_general(
            y_ref[...], z_ref[...], (((1,), (1,)), ((), ())),
            preferred_element_type=jnp.float32)

    def _epi(src_ref):
        dots = src_ref[...]
        denom = jnp.maximum(ny_ref[jax.lax.rem(je, 2)] * nx_ref[ie], 1e-8)
        sim = dots / denom

        rows = jax.lax.broadcasted_iota(jnp.int32, (_BM, _BN), 0)
        cols = jax.lax.broadcasted_iota(jnp.int32, (_BM, _BN), 1)
        cr = cols - rows

        dnew = jnp.sum(jnp.where(cr == 0, sim, 0.0), axis=1, keepdims=True)
        d = jnp.where(se == 0, dnew, d_ref[...])
        d_ref[...] = d

        tie = cr < (je * _BM - ie * _BN)
        beats = (sim > d) | ((sim == d) & tie)
        prev = jnp.where(se > 0, acc_ref[...], 0.0)
        acc_ref[...] = prev + beats.astype(jnp.float32)

    @pl.when(jax.lax.rem(t, 2) == 0)
    def _():
        _mm(da_ref)
        _epi(db_ref)

    @pl.when(jax.lax.rem(t, 2) == 1)
    def _():
        _mm(db_ref)
        _epi(da_ref)

    @pl.when((t >= 1) & (se == _IT - 1))
    def _():
        cnt = jax.lax.dot_general(
            acc_ref[...], ones_col, (((1,), (0,)), ((), ())),
            preferred_element_type=jnp.float32)
        t1 = jnp.where(je == 0, 0.0, t1_ref[...])
        t10 = jnp.where(je == 0, 0.0, t10_ref[...])
        t1_ref[...] = t1 + jnp.sum(
            (cnt == 0.0).astype(jnp.float32), keepdims=True) * (1.0 / _B)
        t10_ref[...] = t10 + jnp.sum(
            (cnt < 10.0).astype(jnp.float32), keepdims=True) * (1.0 / _B)


def _ymap(t):
    tm = jnp.minimum(t, _N - 1)
    return (tm // _IT, 0)


def _zmap(t):
    tm = jnp.minimum(t, _N - 1)
    jm = tm // _IT
    sm = tm - jm * _IT
    return (jax.lax.rem(jm + sm, _IT), 0)


def kernel(Z, Y):
    t1, t10 = pl.pallas_call(
        _body,
        grid=(_N + 1,),
        in_specs=[
            pl.BlockSpec((_BM, _K), _ymap),
            pl.BlockSpec((_BN, _K), _zmap),
        ],
        out_specs=[
            pl.BlockSpec((1, 1), lambda t: (0, 0)),
            pl.BlockSpec((1, 1), lambda t: (0, 0)),
        ],
        out_shape=[
            jax.ShapeDtypeStruct((1, 1), jnp.float32),
            jax.ShapeDtypeStruct((1, 1), jnp.float32),
        ],
        scratch_shapes=[
            pltpu.VMEM((_BM, _BN), jnp.float32),
            pltpu.VMEM((_BM, _BN), jnp.float32),
            pltpu.VMEM((_BM, 1), jnp.float32),
            pltpu.VMEM((2, _BM, 1), jnp.float32),
            pltpu.VMEM((_IT, 1, _BN), jnp.float32),
            pltpu.VMEM((_BM, _BN), jnp.float32),
        ],
        compiler_params=pltpu.CompilerParams(
            dimension_semantics=("arbitrary",)),
    )(Y, Z)
    return (t1[0, 0], t10[0, 0])


# speedup vs baseline: 1.1111x; 1.1111x over previous
import jax
import jax.numpy as jnp
from jax.experimental import pallas as pl
from jax.experimental.pallas import tpu as pltpu

_B = 4096
_K = 1024
_BM = 1024
_BN = 1024
_JT = _B // _BM
_IT = _B // _BN
_N = _JT * _IT


def _body(y_ref, z_ref, t1_ref, t10_ref,
          dbuf_ref, d_ref, ny_ref, nx_ref, acc_ref):
    t = pl.program_id(0)
    ones_row = jnp.ones((1, _K), dtype=jnp.float32)
    ones_col = jnp.ones((_BN, 1), dtype=jnp.bfloat16)

    @pl.when(t < _N)
    def _mm():
        jm = t // _IT
        sm = jax.lax.rem(t, _IT)
        im = jax.lax.rem(jm + sm, _IT)
        y = y_ref[...]
        z = z_ref[...]
        dots = jax.lax.dot_general(
            y, z, (((1,), (1,)), ((), ())),
            preferred_element_type=jnp.float32)
        dbuf_ref[jax.lax.rem(t, 2)] = dots

        @pl.when(sm == 0)
        def _():
            ny2 = jax.lax.dot_general(
                y * y, ones_row, (((1,), (1,)), ((), ())),
                preferred_element_type=jnp.float32)
            ny_ref[jax.lax.rem(jm, 2)] = jnp.sqrt(ny2)

        @pl.when(jm == 0)
        def _():
            nx2 = jax.lax.dot_general(
                ones_row, z * z, (((1,), (1,)), ((), ())),
                preferred_element_type=jnp.float32)
            nx_ref[im] = jnp.sqrt(nx2)

    @pl.when(t >= 1)
    def _epi():
        te = t - 1
        je = te // _IT
        se = jax.lax.rem(te, _IT)
        ie = jax.lax.rem(je + se, _IT)
        dots = dbuf_ref[jax.lax.rem(te, 2)]

        denom = jnp.maximum(ny_ref[jax.lax.rem(je, 2)] * nx_ref[ie], 1e-8)
        sim = dots / denom

        @pl.when(se == 0)
        def _():
            rows = jax.lax.broadcasted_iota(jnp.int32, (_BM, _BN), 0)
            cols = jax.lax.broadcasted_iota(jnp.int32, (_BM, _BN), 1)
            masked = jnp.where(rows == cols, sim, 0.0)
            d_ref[...] = jnp.sum(masked, axis=1, keepdims=True)
            d0 = d_ref[...]
            beats = (sim > d0) | ((sim == d0) & (cols < rows))
            acc_ref[...] = beats.astype(jnp.bfloat16)

        d = d_ref[...]

        @pl.when((se > 0) & (je + se < _IT))
        def _():
            acc_ref[...] += (sim > d).astype(jnp.bfloat16)

        @pl.when(je + se >= _IT)
        def _():
            acc_ref[...] += (sim >= d).astype(jnp.bfloat16)

        @pl.when(te == 0)
        def _():
            t1_ref[...] = jnp.zeros_like(t1_ref)
            t10_ref[...] = jnp.zeros_like(t10_ref)

        @pl.when(se == _IT - 1)
        def _():
            cnt = jax.lax.dot_general(
                acc_ref[...], ones_col, (((1,), (0,)), ((), ())),
                preferred_element_type=jnp.float32)
            t1_ref[...] += jnp.sum(
                (cnt == 0.0).astype(jnp.float32), keepdims=True) * (1.0 / _B)
            t10_ref[...] += jnp.sum(
                (cnt < 10.0).astype(jnp.float32), keepdims=True) * (1.0 / _B)


def _ymap(t):
    tm = jnp.minimum(t, _N - 1)
    return (tm // _IT, 0)


def _zmap(t):
    tm = jnp.minimum(t, _N - 1)
    jm = tm // _IT
    sm = tm - jm * _IT
    return (jax.lax.rem(jm + sm, _IT), 0)


def kernel(Z, Y):
    t1, t10 = pl.pallas_call(
        _body,
        grid=(_N + 1,),
        in_specs=[
            pl.BlockSpec((_BM, _K), _ymap),
            pl.BlockSpec((_BN, _K), _zmap),
        ],
        out_specs=[
            pl.BlockSpec((1, 1), lambda t: (0, 0)),
            pl.BlockSpec((1, 1), lambda t: (0, 0)),
        ],
        out_shape=[
            jax.ShapeDtypeStruct((1, 1), jnp.float32),
            jax.ShapeDtypeStruct((1, 1), jnp.float32),
        ],
        scratch_shapes=[
            pltpu.VMEM((2, _BM, _BN), jnp.float32),
            pltpu.VMEM((_BM, 1), jnp.float32),
            pltpu.VMEM((2, _BM, 1), jnp.float32),
            pltpu.VMEM((_IT, 1, _BN), jnp.float32),
            pltpu.VMEM((_BM, _BN), jnp.bfloat16),
        ],
        compiler_params=pltpu.CompilerParams(
            dimension_semantics=("arbitrary",)),
    )(Y, Z)
    return (t1[0, 0], t10[0, 0])


# speedup vs baseline: 1.1199x; 1.0079x over previous
import jax
import jax.numpy as jnp
from jax.experimental import pallas as pl
from jax.experimental.pallas import tpu as pltpu

_B = 4096
_K = 1024
_BM = 1024
_BN = 1024
_JT = _B // _BM
_IT = _B // _BN
_N = _JT * _IT


def _body(y_ref, z_ref, t1_ref, t10_ref,
          dbuf_ref, d_ref, ny_ref, nx_ref, acc_ref):
    t = pl.program_id(0)
    ones_row = jnp.ones((1, _K), dtype=jnp.float32)
    ones_col = jnp.ones((_BN, 1), dtype=jnp.bfloat16)

    @pl.when(t < _N)
    def _mm():
        jm = t // _IT
        sm = jax.lax.rem(t, _IT)
        im = jax.lax.rem(jm + sm, _IT)
        y = y_ref[...]
        z = z_ref[...]
        dots = jax.lax.dot_general(
            y, z, (((1,), (1,)), ((), ())),
            preferred_element_type=jnp.float32)
        dbuf_ref[jax.lax.rem(t, 2)] = dots

        @pl.when(sm == 0)
        def _():
            ny2 = jax.lax.dot_general(
                y * y, ones_row, (((1,), (1,)), ((), ())),
                preferred_element_type=jnp.float32)
            ny_ref[jax.lax.rem(jm, 2)] = jnp.sqrt(ny2)

        @pl.when(jm == 0)
        def _():
            nx2 = jax.lax.dot_general(
                ones_row, z * z, (((1,), (1,)), ((), ())),
                preferred_element_type=jnp.float32)
            nx_ref[im] = jnp.sqrt(nx2)

    @pl.when(t >= 1)
    def _epi():
        te = t - 1
        je = te // _IT
        se = jax.lax.rem(te, _IT)
        ie = jax.lax.rem(je + se, _IT)
        dots = dbuf_ref[jax.lax.rem(te, 2)]

        denom = jnp.maximum(ny_ref[jax.lax.rem(je, 2)] * nx_ref[ie], 1e-8)
        sim = dots / denom

        @pl.when(se == 0)
        def _():
            rows = jax.lax.broadcasted_iota(jnp.int32, (_BM, _BN), 0)
            cols = jax.lax.broadcasted_iota(jnp.int32, (_BM, _BN), 1)
            masked = jnp.where(rows == cols, sim, 0.0)
            d_ref[...] = jnp.sum(masked, axis=1, keepdims=True)
            d0 = d_ref[...]
            beats = (sim > d0) | ((sim == d0) & (cols < rows))
            acc_ref[...] = beats.astype(jnp.bfloat16)

        d = d_ref[...]

        @pl.when((se > 0) & (je + se < _IT))
        def _():
            acc_ref[...] += (sim > d).astype(jnp.bfloat16)

        @pl.when(je + se >= _IT)
        def _():
            acc_ref[...] += (sim >= d).astype(jnp.bfloat16)

        @pl.when(te == 0)
        def _():
            t1_ref[...] = jnp.zeros_like(t1_ref)
            t10_ref[...] = jnp.zeros_like(t10_ref)

        @pl.when(se == _IT - 1)
        def _():
            cnt = jax.lax.dot_general(
                acc_ref[...], ones_col, (((1,), (0,)), ((), ())),
                preferred_element_type=jnp.float32)
            t1_ref[...] += jnp.sum(
                (cnt == 0.0).astype(jnp.float32), keepdims=True) * (1.0 / _B)
            t10_ref[...] += jnp.sum(
                (cnt < 10.0).astype(jnp.float32), keepdims=True) * (1.0 / _B)

    @pl.when(t == _N - 1)
    def _tail():
        je2 = _JT - 1
        ie2 = (je2 + (_IT - 1)) % _IT
        dots2 = dbuf_ref[(_N - 1) % 2]
        denom2 = jnp.maximum(
            ny_ref[je2 % 2] * nx_ref[ie2], 1e-8)
        sim2 = dots2 / denom2
        acc2 = acc_ref[...] + (sim2 >= d_ref[...]).astype(jnp.bfloat16)
        cnt = jax.lax.dot_general(
            acc2, ones_col, (((1,), (0,)), ((), ())),
            preferred_element_type=jnp.float32)
        t1_ref[...] += jnp.sum(
            (cnt == 0.0).astype(jnp.float32), keepdims=True) * (1.0 / _B)
        t10_ref[...] += jnp.sum(
            (cnt < 10.0).astype(jnp.float32), keepdims=True) * (1.0 / _B)


def _ymap(t):
    tm = jnp.minimum(t, _N - 1)
    return (tm // _IT, 0)


def _zmap(t):
    tm = jnp.minimum(t, _N - 1)
    jm = tm // _IT
    sm = tm - jm * _IT
    return (jax.lax.rem(jm + sm, _IT), 0)


def kernel(Z, Y):
    t1, t10 = pl.pallas_call(
        _body,
        grid=(_N,),
        in_specs=[
            pl.BlockSpec((_BM, _K), _ymap),
            pl.BlockSpec((_BN, _K), _zmap),
        ],
        out_specs=[
            pl.BlockSpec((1, 1), lambda t: (0, 0)),
            pl.BlockSpec((1, 1), lambda t: (0, 0)),
        ],
        out_shape=[
            jax.ShapeDtypeStruct((1, 1), jnp.float32),
            jax.ShapeDtypeStruct((1, 1), jnp.float32),
        ],
        scratch_shapes=[
            pltpu.VMEM((2, _BM, _BN), jnp.float32),
            pltpu.VMEM((_BM, 1), jnp.float32),
            pltpu.VMEM((2, _BM, 1), jnp.float32),
            pltpu.VMEM((_IT, 1, _BN), jnp.float32),
            pltpu.VMEM((_BM, _BN), jnp.bfloat16),
        ],
        compiler_params=pltpu.CompilerParams(
            dimension_semantics=("arbitrary",)),
    )(Y, Z)
    return (t1[0, 0], t10[0, 0])
